# baseline (speedup 1.0000x reference)
import jax
import jax.numpy as jnp
from jax.experimental import pallas as pl
from jax.experimental.pallas import tpu as pltpu

N_ROWS = 16384
N_COLS = 4096
TILE_C = 128
K = (N_ROWS - 1) // 2

CH = 512
NCH = N_ROWS // CH


def _to_key(f):
    s = jax.lax.bitcast_convert_type(f, jnp.int32)
    return s ^ ((s >> 31) & 0x7FFFFFFF)


def _from_key(k):
    s = k ^ ((k >> 31) & 0x7FFFFFFF)
    return jax.lax.bitcast_convert_type(s, jnp.float32)


CB = 128


def _count_below(ref, q16):
    def chunk(c, acc):
        v = ref[pl.ds(c * CB, CB), :]
        return acc + jnp.where(v < q16, jnp.int16(-1), jnp.int16(0))

    acc = jax.lax.fori_loop(
        0, N_ROWS // CB, chunk, jnp.zeros((CB, TILE_C), jnp.int16), unroll=2)
    return -jnp.sum(acc.astype(jnp.int32), axis=0, keepdims=True)


def _median_pair_body(x_ref, o_ref, kt_ref, kl_ref):
    def stage0(c, _):
        key = _to_key(x_ref[pl.ds(c * CH, CH), :])
        kt_ref[pl.ds(c * CH, CH), :] = (key >> 16).astype(jnp.int16)
        kl_ref[pl.ds(c * CH, CH), :] = ((key & 0xFFFF) ^ 0x8000).astype(jnp.int16)
        return 0

    jax.lax.fori_loop(0, NCH, stage0, 0, unroll=True)

    def step_a(i, p):
        q = p + jax.lax.shift_left(jnp.ones((), jnp.int32), 15 - i)
        return jnp.where(_count_below(kt_ref, q.astype(jnp.int16)) <= K, q, p)

    p16 = jax.lax.fori_loop(
        0, 16, step_a, jnp.full((1, TILE_C), -32768, dtype=jnp.int32))
    p16_16 = p16.astype(jnp.int16)
    c0 = _count_below(kt_ref, p16_16)

    def trans(c, _):
        kt = kt_ref[pl.ds(c * CH, CH), :]
        kl = kl_ref[pl.ds(c * CH, CH), :]
        kl_ref[pl.ds(c * CH, CH), :] = jnp.where(kt == p16_16, kl, jnp.int16(32767))
        return 0

    jax.lax.fori_loop(0, NCH, trans, 0, unroll=True)

    kb = (K - c0).astype(jnp.int32)

    def step_b(i, p):
        q = p + jax.lax.shift_left(jnp.ones((), jnp.int32), 15 - i)
        return jnp.where(_count_below(kl_ref, q.astype(jnp.int16)) <= kb, q, p)

    plow = jax.lax.fori_loop(
        0, 16, step_b, jnp.full((1, TILE_C), -32768, dtype=jnp.int32))

    key_lo = (p16 << 16) | ((plow & 0xFFFF) ^ 0x8000)

    def tail(c, carry):
        cnt, mn = carry
        key = _to_key(x_ref[pl.ds(c * CB, CB), :])
        cnt = cnt + jnp.where(key <= key_lo, -1, 0)
        mn = jnp.minimum(mn, jnp.where(key > key_lo, key, 2147483647))
        return cnt, mn

    cnt0 = jnp.zeros((CB, TILE_C), jnp.int32)
    mn0 = jnp.full((CB, TILE_C), 2147483647, dtype=jnp.int32)
    cntb, mnb = jax.lax.fori_loop(0, N_ROWS // CB, tail, (cnt0, mn0), unroll=2)
    cnt_le = -jnp.sum(cntb, axis=0, keepdims=True)
    mn_above = jnp.min(mnb, axis=0, keepdims=True)

    key_hi = jnp.where(cnt_le >= K + 2, key_lo, mn_above)

    lower = _from_key(key_lo)
    upper = _from_key(key_hi)
    o_ref[...] = jnp.abs(lower - (lower + upper) * 0.5)


@jax.jit
def kernel(x):
    out2d = pl.pallas_call(
        _median_pair_body,
        grid=(N_COLS // TILE_C,),
        in_specs=[pl.BlockSpec((N_ROWS, TILE_C), lambda i: (0, i))],
        out_specs=pl.BlockSpec((1, TILE_C), lambda i: (0, i)),
        out_shape=jax.ShapeDtypeStruct((1, N_COLS), jnp.float32),
        scratch_shapes=[
            pltpu.VMEM((N_ROWS, TILE_C), jnp.int16),
            pltpu.VMEM((N_ROWS, TILE_C), jnp.int16),
        ],
    )(x)
    return out2d[0]
